# Initial kernel scaffold; baseline (speedup 1.0000x reference)
import jax
import jax.numpy as jnp
from jax.experimental import pallas as pl


def kernel(x, edge_index, batch, W1_0, b1_0, W2_0, b2_0, W1_1, b1_1, W2_1, b2_1, W1_2, b1_2, W2_2, b2_2):
    raise NotImplementedError("write your pallas kernel here")


# speedup vs baseline: 5.7469x; 5.7469x over previous
import functools

import jax
import jax.numpy as jnp
from jax import lax
from jax.experimental import pallas as pl
from jax.experimental.pallas import tpu as pltpu
from jax.experimental.pallas import tpu_sc as plsc

N = 10000
E = 320000
D = 128
G = 64

NC = 2
NS = 16
NW = NC * NS

CHUNK = 128
N_CHUNKS = E // CHUNK
BASE_CHUNKS = N_CHUNKS // NW
N_EXTRA = N_CHUNKS - BASE_CHUNKS * NW

ROWS_MAIN = (N // NS) // 8 * 8
TAIL_ROWS = N - NS * ROWS_MAIN
ZR = 208
BT = 1000
NB = N // BT


def _sc_agg_body(h_hbm, src_hbm, dst_hbm, out_hbm, sidx, didx, rows, zbuf, agg_sh, sem):
    c = lax.axis_index("c")
    s = lax.axis_index("s")
    w = c * NS + s

    def _zero_row(i, carry):
        for j in range(8):
            zbuf[i, pl.ds(j * 16, 16)] = jnp.zeros((16,), jnp.float32)
        return carry

    lax.fori_loop(0, ZR, _zero_row, 0)
    for k in range(ROWS_MAIN // ZR):
        pltpu.sync_copy(zbuf, agg_sh.at[pl.ds(s * ROWS_MAIN + k * ZR, ZR)])

    @pl.when(s == NS - 1)
    def _():
        pltpu.sync_copy(zbuf.at[pl.ds(0, TAIL_ROWS)], agg_sh.at[pl.ds(NS * ROWS_MAIN, TAIL_ROWS)])

    plsc.subcore_barrier()

    nchunks = BASE_CHUNKS + (w < N_EXTRA).astype(jnp.int32)
    base = w * BASE_CHUNKS
    extra_off = (NW * BASE_CHUNKS + w) * CHUNK

    def _edge_chunk(i, carry):
        off = lax.select(i < BASE_CHUNKS, (base + i) * CHUNK, extra_off)
        pltpu.sync_copy(src_hbm.at[pl.ds(off, CHUNK)], sidx)
        pltpu.sync_copy(dst_hbm.at[pl.ds(off, CHUNK)], didx)
        pltpu.async_copy(h_hbm.at[sidx], rows, sem).wait()
        pltpu.sync_copy(rows, agg_sh.at[didx], add=True)
        return carry

    lax.fori_loop(0, nchunks, _edge_chunk, 0)
    plsc.subcore_barrier()

    pltpu.sync_copy(
        agg_sh.at[pl.ds(s * ROWS_MAIN, ROWS_MAIN)],
        out_hbm.at[c, pl.ds(s * ROWS_MAIN, ROWS_MAIN)],
    )

    @pl.when(s == NS - 1)
    def _():
        pltpu.sync_copy(
            agg_sh.at[pl.ds(NS * ROWS_MAIN, TAIL_ROWS)],
            out_hbm.at[c, pl.ds(NS * ROWS_MAIN, TAIL_ROWS)],
        )


def _sc_agg(h, src, dst):
    mesh = plsc.VectorSubcoreMesh(core_axis_name="c", subcore_axis_name="s")
    return pl.kernel(
        _sc_agg_body,
        out_type=jax.ShapeDtypeStruct((NC, N, D), jnp.float32),
        mesh=mesh,
        scratch_types=[
            pltpu.VMEM((CHUNK,), jnp.int32),
            pltpu.VMEM((CHUNK,), jnp.int32),
            pltpu.VMEM((CHUNK, D), jnp.float32),
            pltpu.VMEM((ZR, D), jnp.float32),
            pltpu.VMEM_SHARED((N, D), jnp.float32),
            pltpu.SemaphoreType.DMA,
        ],
    )(h, src, dst)


def _mlp_body(h_ref, a_ref, w1_ref, b1_ref, w2_ref, b2_ref, o_ref):
    z = h_ref[...] + a_ref[0] + a_ref[1]
    z = jnp.maximum(jnp.dot(z, w1_ref[...], preferred_element_type=jnp.float32) + b1_ref[...], 0.0)
    z = jnp.dot(z, w2_ref[...], preferred_element_type=jnp.float32) + b2_ref[...]
    o_ref[...] = jnp.maximum(z, 0.0)


def _mlp(h, agg, W1, b1, W2, b2):
    return pl.pallas_call(
        _mlp_body,
        grid=(NB,),
        in_specs=[
            pl.BlockSpec((BT, D), lambda i: (i, 0)),
            pl.BlockSpec((NC, BT, D), lambda i: (0, i, 0)),
            pl.BlockSpec((D, D), lambda i: (0, 0)),
            pl.BlockSpec((1, D), lambda i: (0, 0)),
            pl.BlockSpec((D, D), lambda i: (0, 0)),
            pl.BlockSpec((1, D), lambda i: (0, 0)),
        ],
        out_specs=pl.BlockSpec((BT, D), lambda i: (i, 0)),
        out_shape=jax.ShapeDtypeStruct((N, D), jnp.float32),
    )(h, agg, W1, b1.reshape(1, D), W2, b2.reshape(1, D))


def _mlp_pool_body(h_ref, a_ref, w1_ref, b1_ref, w2_ref, b2_ref, bt_ref, o_ref):
    i = pl.program_id(0)
    z = h_ref[...] + a_ref[0] + a_ref[1]
    z = jnp.maximum(jnp.dot(z, w1_ref[...], preferred_element_type=jnp.float32) + b1_ref[...], 0.0)
    z = jnp.dot(z, w2_ref[...], preferred_element_type=jnp.float32) + b2_ref[...]
    hb = jnp.maximum(z, 0.0)
    b = bt_ref[0, 0, :]
    onehot = (lax.broadcasted_iota(jnp.int32, (G, BT), 0) == b[None, :]).astype(jnp.float32)
    part = jnp.dot(onehot, hb, preferred_element_type=jnp.float32)

    @pl.when(i == 0)
    def _():
        o_ref[...] = part

    @pl.when(i > 0)
    def _():
        o_ref[...] += part


def _mlp_pool(h, agg, W1, b1, W2, b2, batch):
    return pl.pallas_call(
        _mlp_pool_body,
        grid=(NB,),
        in_specs=[
            pl.BlockSpec((BT, D), lambda i: (i, 0)),
            pl.BlockSpec((NC, BT, D), lambda i: (0, i, 0)),
            pl.BlockSpec((D, D), lambda i: (0, 0)),
            pl.BlockSpec((1, D), lambda i: (0, 0)),
            pl.BlockSpec((D, D), lambda i: (0, 0)),
            pl.BlockSpec((1, D), lambda i: (0, 0)),
            pl.BlockSpec((1, 1, BT), lambda i: (i, 0, 0)),
        ],
        out_specs=pl.BlockSpec((G, D), lambda i: (0, 0)),
        out_shape=jax.ShapeDtypeStruct((G, D), jnp.float32),
        compiler_params=pltpu.CompilerParams(dimension_semantics=("arbitrary",)),
    )(h, agg, W1, b1.reshape(1, D), W2, b2.reshape(1, D), batch.reshape(NB, 1, BT))


def kernel(x, edge_index, batch, W1_0, b1_0, W2_0, b2_0, W1_1, b1_1, W2_1, b2_1, W1_2, b1_2, W2_2, b2_2):
    src = edge_index[0]
    dst = edge_index[1]
    params = [(W1_0, b1_0, W2_0, b2_0), (W1_1, b1_1, W2_1, b2_1), (W1_2, b1_2, W2_2, b2_2)]
    h = x
    for li, (W1, b1, W2, b2) in enumerate(params):
        agg = _sc_agg(h, src, dst)
        if li < 2:
            h = _mlp(h, agg, W1, b1, W2, b2)
        else:
            out = _mlp_pool(h, agg, W1, b1, W2, b2, batch)
    return out


# speedup vs baseline: 7.6939x; 1.3388x over previous
import functools

import numpy as np

import jax
import jax.numpy as jnp
from jax import lax
from jax.experimental import pallas as pl
from jax.experimental.pallas import tpu as pltpu
from jax.experimental.pallas import tpu_sc as plsc

N = 10000
E = 320000
D = 128
G = 64

NC = 2
NS = 16
NW = NC * NS

CHUNK = 128
N_CHUNKS = E // CHUNK
BASE_CHUNKS = N_CHUNKS // NW
N_EXTRA = N_CHUNKS - BASE_CHUNKS * NW

ROWS_MAIN = (N // NS) // 8 * 8
TAIL_ROWS = N - NS * ROWS_MAIN

NBLK = 3
BPC = BASE_CHUNKS // NBLK
PAIRS = BPC // 2
BT = 1000
NB = N // BT

_CHUNK_IDS = np.array(
    [
        list(range(w * BASE_CHUNKS, (w + 1) * BASE_CHUNKS))
        + [NW * BASE_CHUNKS + w if w < N_EXTRA else 0]
        + [0] * (BPC - 1)
        for w in range(NW)
    ],
    dtype=np.int32,
)


def _sc_agg_body(h_hbm, idx_hbm, out_hbm, idx_u, idx_v, rows_a, rows_b,
                 agg_sh, sem_i, sem_ga, sem_gb, sem_sa, sem_sb):
    c = lax.axis_index("c")
    s = lax.axis_index("s")
    w = c * NS + s

    def _zero_row(i, carry):
        for j in range(D // 16):
            rows_a[i, pl.ds(j * 16, 16)] = jnp.zeros((16,), jnp.float32)
        return carry

    lax.fori_loop(0, CHUNK, _zero_row, 0)
    done = 0
    while done < ROWS_MAIN:
        step = min(CHUNK, ROWS_MAIN - done)
        pltpu.sync_copy(rows_a.at[pl.ds(0, step)],
                        agg_sh.at[pl.ds(s * ROWS_MAIN + done, step)])
        done += step

    @pl.when(s == NS - 1)
    def _():
        pltpu.sync_copy(rows_a.at[pl.ds(0, TAIL_ROWS)], agg_sh.at[pl.ds(NS * ROWS_MAIN, TAIL_ROWS)])

    plsc.subcore_barrier()

    def _gather(idx_row, buf, sem):
        pltpu.async_copy(h_hbm.at[idx_row], buf, sem)

    def _scatter(idx_row, buf, sem):
        pltpu.async_copy(buf, agg_sh.at[idx_row], sem, add=True)

    def _wait_g(idx_row, buf, sem):
        pltpu.make_async_copy(h_hbm.at[idx_row], buf, sem).wait()

    def _wait_s(idx_row, buf, sem):
        pltpu.make_async_copy(buf, agg_sh.at[idx_row], sem).wait()

    pltpu.sync_copy(idx_hbm.at[w, 0], idx_u)
    _gather(idx_u.at[0], rows_a, sem_ga)

    for b in range(NBLK):
        ib = (idx_u, idx_v)[b % 2]
        nb = (idx_v, idx_u)[b % 2]

        def _pair(k, carry, ib=ib, nb=nb, first=(b == 0)):
            def _drain_b():
                _wait_s(ib.at[3], rows_b, sem_sb)

            if first:
                pl.when(k > 0)(_drain_b)
            else:
                _drain_b()
            _gather(ib.at[4 * k + 2], rows_b, sem_gb)
            _wait_g(ib.at[4 * k], rows_a, sem_ga)
            _scatter(ib.at[4 * k + 1], rows_a, sem_sa)
            _wait_g(ib.at[4 * k + 2], rows_b, sem_gb)
            _scatter(ib.at[4 * k + 3], rows_b, sem_sb)
            _wait_s(ib.at[1], rows_a, sem_sa)

            @pl.when(k < PAIRS - 1)
            def _():
                _gather(ib.at[4 * k + 4], rows_a, sem_ga)

            if b < NBLK - 1:
                @pl.when(k == 0)
                def _():
                    pltpu.async_copy(idx_hbm.at[w, b + 1], nb, sem_i)
            return carry

        lax.fori_loop(0, PAIRS, _pair, 0)
        if b < NBLK - 1:
            pltpu.make_async_copy(idx_hbm.at[w, b + 1], nb, sem_i).wait()
            _gather(nb.at[0], rows_a, sem_ga)

    last_ib = (idx_u, idx_v)[(NBLK - 1) % 2]
    _wait_s(last_ib.at[3], rows_b, sem_sb)

    @pl.when(w < N_EXTRA)
    def _():
        pltpu.sync_copy(idx_hbm.at[w, NBLK], idx_v)
        _gather(idx_v.at[0], rows_a, sem_ga)
        _wait_g(idx_v.at[0], rows_a, sem_ga)
        _scatter(idx_v.at[1], rows_a, sem_sa)
        _wait_s(idx_v.at[1], rows_a, sem_sa)

    plsc.subcore_barrier()

    pltpu.sync_copy(
        agg_sh.at[pl.ds(s * ROWS_MAIN, ROWS_MAIN)],
        out_hbm.at[c, pl.ds(s * ROWS_MAIN, ROWS_MAIN)],
    )

    @pl.when(s == NS - 1)
    def _():
        pltpu.sync_copy(
            agg_sh.at[pl.ds(NS * ROWS_MAIN, TAIL_ROWS)],
            out_hbm.at[c, pl.ds(NS * ROWS_MAIN, TAIL_ROWS)],
        )


def _sc_agg(h, idx4):
    mesh = plsc.VectorSubcoreMesh(core_axis_name="c", subcore_axis_name="s")
    return pl.kernel(
        _sc_agg_body,
        out_type=jax.ShapeDtypeStruct((NC, N, D), jnp.float32),
        mesh=mesh,
        scratch_types=[
            pltpu.VMEM((2 * BPC, CHUNK), jnp.int32),
            pltpu.VMEM((2 * BPC, CHUNK), jnp.int32),
            pltpu.VMEM((CHUNK, D), jnp.float32),
            pltpu.VMEM((CHUNK, D), jnp.float32),
            pltpu.VMEM_SHARED((N, D), jnp.float32),
            pltpu.SemaphoreType.DMA,
            pltpu.SemaphoreType.DMA,
            pltpu.SemaphoreType.DMA,
            pltpu.SemaphoreType.DMA,
            pltpu.SemaphoreType.DMA,
        ],
    )(h, idx4)


def _mlp_body(h_ref, a_ref, w1_ref, b1_ref, w2_ref, b2_ref, o_ref):
    z = h_ref[...] + a_ref[0] + a_ref[1]
    z = jnp.maximum(jnp.dot(z, w1_ref[...], preferred_element_type=jnp.float32) + b1_ref[...], 0.0)
    z = jnp.dot(z, w2_ref[...], preferred_element_type=jnp.float32) + b2_ref[...]
    o_ref[...] = jnp.maximum(z, 0.0)


def _mlp(h, agg, W1, b1, W2, b2):
    return pl.pallas_call(
        _mlp_body,
        grid=(NB,),
        in_specs=[
            pl.BlockSpec((BT, D), lambda i: (i, 0)),
            pl.BlockSpec((NC, BT, D), lambda i: (0, i, 0)),
            pl.BlockSpec((D, D), lambda i: (0, 0)),
            pl.BlockSpec((1, D), lambda i: (0, 0)),
            pl.BlockSpec((D, D), lambda i: (0, 0)),
            pl.BlockSpec((1, D), lambda i: (0, 0)),
        ],
        out_specs=pl.BlockSpec((BT, D), lambda i: (i, 0)),
        out_shape=jax.ShapeDtypeStruct((N, D), jnp.float32),
    )(h, agg, W1, b1.reshape(1, D), W2, b2.reshape(1, D))


def _mlp_pool_body(h_ref, a_ref, w1_ref, b1_ref, w2_ref, b2_ref, bt_ref, o_ref):
    i = pl.program_id(0)
    z = h_ref[...] + a_ref[0] + a_ref[1]
    z = jnp.maximum(jnp.dot(z, w1_ref[...], preferred_element_type=jnp.float32) + b1_ref[...], 0.0)
    z = jnp.dot(z, w2_ref[...], preferred_element_type=jnp.float32) + b2_ref[...]
    hb = jnp.maximum(z, 0.0)
    b = bt_ref[0, 0, :]
    onehot = (lax.broadcasted_iota(jnp.int32, (G, BT), 0) == b[None, :]).astype(jnp.float32)
    part = jnp.dot(onehot, hb, preferred_element_type=jnp.float32)

    @pl.when(i == 0)
    def _():
        o_ref[...] = part

    @pl.when(i > 0)
    def _():
        o_ref[...] += part


def _mlp_pool(h, agg, W1, b1, W2, b2, batch):
    return pl.pallas_call(
        _mlp_pool_body,
        grid=(NB,),
        in_specs=[
            pl.BlockSpec((BT, D), lambda i: (i, 0)),
            pl.BlockSpec((NC, BT, D), lambda i: (0, i, 0)),
            pl.BlockSpec((D, D), lambda i: (0, 0)),
            pl.BlockSpec((1, D), lambda i: (0, 0)),
            pl.BlockSpec((D, D), lambda i: (0, 0)),
            pl.BlockSpec((1, D), lambda i: (0, 0)),
            pl.BlockSpec((1, 1, BT), lambda i: (i, 0, 0)),
        ],
        out_specs=pl.BlockSpec((G, D), lambda i: (0, 0)),
        out_shape=jax.ShapeDtypeStruct((G, D), jnp.float32),
        compiler_params=pltpu.CompilerParams(dimension_semantics=("arbitrary",)),
    )(h, agg, W1, b1.reshape(1, D), W2, b2.reshape(1, D), batch.reshape(NB, 1, BT))


def kernel(x, edge_index, batch, W1_0, b1_0, W2_0, b2_0, W1_1, b1_1, W2_1, b2_1, W1_2, b1_2, W2_2, b2_2):
    srcsel = jnp.take(edge_index[0].reshape(N_CHUNKS, CHUNK), _CHUNK_IDS.reshape(-1), axis=0)
    dstsel = jnp.take(edge_index[1].reshape(N_CHUNKS, CHUNK), _CHUNK_IDS.reshape(-1), axis=0)
    idx4 = jnp.stack([srcsel, dstsel], axis=1).reshape(
        NW, (NBLK + 1) * BPC, 2, CHUNK).reshape(NW, NBLK + 1, 2 * BPC, CHUNK)
    params = [(W1_0, b1_0, W2_0, b2_0), (W1_1, b1_1, W2_1, b2_1), (W1_2, b1_2, W2_2, b2_2)]
    h = x
    for li, (W1, b1, W2, b2) in enumerate(params):
        agg = _sc_agg(h, idx4)
        if li < 2:
            h = _mlp(h, agg, W1, b1, W2, b2)
        else:
            out = _mlp_pool(h, agg, W1, b1, W2, b2, batch)
    return out


# speedup vs baseline: 8.9235x; 1.1598x over previous
import functools

import numpy as np

import jax
import jax.numpy as jnp
from jax import lax
from jax.experimental import pallas as pl
from jax.experimental.pallas import tpu as pltpu
from jax.experimental.pallas import tpu_sc as plsc

N = 10000
E = 320000
D = 128
G = 64

NC = 2
NS = 16
NW = NC * NS

CHUNK = 128
N_CHUNKS = E // CHUNK
BASE_CHUNKS = N_CHUNKS // NW
N_EXTRA = N_CHUNKS - BASE_CHUNKS * NW

ROWS_MAIN = (N // NS) // 8 * 8
TAIL_ROWS = N - NS * ROWS_MAIN

NBLK = 3
BPC = BASE_CHUNKS // NBLK
PAIRS = BPC // 2
BT = 1000
NB = N // BT


def _sc_agg_body(h_hbm, srcb_hbm, dstb_hbm, ext_hbm, out_hbm, s_u, s_v, d_u, d_v,
                 rows_a, rows_b, agg_sh, sem_i, sem_ga, sem_gb, sem_sa, sem_sb):
    c = lax.axis_index("c")
    s = lax.axis_index("s")
    w = c * NS + s

    def _zero_row(i, carry):
        for j in range(D // 16):
            rows_a[i, pl.ds(j * 16, 16)] = jnp.zeros((16,), jnp.float32)
        return carry

    lax.fori_loop(0, CHUNK, _zero_row, 0)
    done = 0
    while done < ROWS_MAIN:
        step = min(CHUNK, ROWS_MAIN - done)
        pltpu.sync_copy(rows_a.at[pl.ds(0, step)],
                        agg_sh.at[pl.ds(s * ROWS_MAIN + done, step)])
        done += step

    @pl.when(s == NS - 1)
    def _():
        pltpu.sync_copy(rows_a.at[pl.ds(0, TAIL_ROWS)], agg_sh.at[pl.ds(NS * ROWS_MAIN, TAIL_ROWS)])

    plsc.subcore_barrier()

    def _gather(idx_row, buf, sem):
        pltpu.async_copy(h_hbm.at[idx_row], buf, sem)

    def _scatter(idx_row, buf, sem):
        pltpu.async_copy(buf, agg_sh.at[idx_row], sem, add=True)

    def _wait_g(idx_row, buf, sem):
        pltpu.make_async_copy(h_hbm.at[idx_row], buf, sem).wait()

    def _wait_s(idx_row, buf, sem):
        pltpu.make_async_copy(buf, agg_sh.at[idx_row], sem).wait()

    pltpu.sync_copy(srcb_hbm.at[w, 0], s_u)
    pltpu.sync_copy(dstb_hbm.at[w, 0], d_u)
    _gather(s_u.at[0], rows_a, sem_ga)

    for b in range(NBLK):
        sb, db = ((s_u, d_u), (s_v, d_v))[b % 2]
        nsb, ndb = ((s_v, d_v), (s_u, d_u))[b % 2]

        def _pair(k, carry, sb=sb, db=db, nsb=nsb, ndb=ndb, b=b):
            def _drain_b():
                _wait_s(db.at[1], rows_b, sem_sb)

            if b == 0:
                pl.when(k > 0)(_drain_b)
            else:
                _drain_b()
            _gather(sb.at[2 * k + 1], rows_b, sem_gb)
            _wait_g(sb.at[2 * k], rows_a, sem_ga)
            _scatter(db.at[2 * k], rows_a, sem_sa)
            _wait_g(sb.at[2 * k + 1], rows_b, sem_gb)
            _scatter(db.at[2 * k + 1], rows_b, sem_sb)
            _wait_s(db.at[0], rows_a, sem_sa)

            @pl.when(k < PAIRS - 1)
            def _():
                _gather(sb.at[2 * k + 2], rows_a, sem_ga)

            if b < NBLK - 1:
                @pl.when(k == 0)
                def _():
                    pltpu.async_copy(srcb_hbm.at[w, b + 1], nsb, sem_i)
                    pltpu.async_copy(dstb_hbm.at[w, b + 1], ndb, sem_i)
            return carry

        lax.fori_loop(0, PAIRS, _pair, 0)
        if b < NBLK - 1:
            pltpu.make_async_copy(srcb_hbm.at[w, b + 1], nsb, sem_i).wait()
            pltpu.make_async_copy(dstb_hbm.at[w, b + 1], ndb, sem_i).wait()
            _gather(nsb.at[0], rows_a, sem_ga)

    _, last_db = ((s_u, d_u), (s_v, d_v))[(NBLK - 1) % 2]
    _wait_s(last_db.at[1], rows_b, sem_sb)

    @pl.when(w < N_EXTRA)
    def _():
        pltpu.sync_copy(ext_hbm.at[w], s_v.at[pl.ds(0, 2)])
        _gather(s_v.at[0], rows_a, sem_ga)
        _wait_g(s_v.at[0], rows_a, sem_ga)
        _scatter(s_v.at[1], rows_a, sem_sa)
        _wait_s(s_v.at[1], rows_a, sem_sa)

    plsc.subcore_barrier()

    pltpu.sync_copy(
        agg_sh.at[pl.ds(s * ROWS_MAIN, ROWS_MAIN)],
        out_hbm.at[c, pl.ds(s * ROWS_MAIN, ROWS_MAIN)],
    )

    @pl.when(s == NS - 1)
    def _():
        pltpu.sync_copy(
            agg_sh.at[pl.ds(NS * ROWS_MAIN, TAIL_ROWS)],
            out_hbm.at[c, pl.ds(NS * ROWS_MAIN, TAIL_ROWS)],
        )


def _sc_agg(h, srcb, dstb, ext):
    mesh = plsc.VectorSubcoreMesh(core_axis_name="c", subcore_axis_name="s")
    return pl.kernel(
        _sc_agg_body,
        out_type=jax.ShapeDtypeStruct((NC, N, D), jnp.float32),
        mesh=mesh,
        scratch_types=[
            pltpu.VMEM((BPC, CHUNK), jnp.int32),
            pltpu.VMEM((BPC, CHUNK), jnp.int32),
            pltpu.VMEM((BPC, CHUNK), jnp.int32),
            pltpu.VMEM((BPC, CHUNK), jnp.int32),
            pltpu.VMEM((CHUNK, D), jnp.float32),
            pltpu.VMEM((CHUNK, D), jnp.float32),
            pltpu.VMEM_SHARED((N, D), jnp.float32),
            pltpu.SemaphoreType.DMA,
            pltpu.SemaphoreType.DMA,
            pltpu.SemaphoreType.DMA,
            pltpu.SemaphoreType.DMA,
            pltpu.SemaphoreType.DMA,
        ],
    )(h, srcb, dstb, ext)


def _mlp_body(h_ref, a_ref, w1_ref, b1_ref, w2_ref, b2_ref, o_ref):
    z = h_ref[...] + a_ref[0] + a_ref[1]
    z = jnp.maximum(jnp.dot(z, w1_ref[...], preferred_element_type=jnp.float32) + b1_ref[...], 0.0)
    z = jnp.dot(z, w2_ref[...], preferred_element_type=jnp.float32) + b2_ref[...]
    o_ref[...] = jnp.maximum(z, 0.0)


def _mlp(h, agg, W1, b1, W2, b2):
    return pl.pallas_call(
        _mlp_body,
        grid=(NB,),
        in_specs=[
            pl.BlockSpec((BT, D), lambda i: (i, 0)),
            pl.BlockSpec((NC, BT, D), lambda i: (0, i, 0)),
            pl.BlockSpec((D, D), lambda i: (0, 0)),
            pl.BlockSpec((1, D), lambda i: (0, 0)),
            pl.BlockSpec((D, D), lambda i: (0, 0)),
            pl.BlockSpec((1, D), lambda i: (0, 0)),
        ],
        out_specs=pl.BlockSpec((BT, D), lambda i: (i, 0)),
        out_shape=jax.ShapeDtypeStruct((N, D), jnp.float32),
    )(h, agg, W1, b1.reshape(1, D), W2, b2.reshape(1, D))


def _mlp_pool_body(h_ref, a_ref, w1_ref, b1_ref, w2_ref, b2_ref, bt_ref, o_ref):
    i = pl.program_id(0)
    z = h_ref[...] + a_ref[0] + a_ref[1]
    z = jnp.maximum(jnp.dot(z, w1_ref[...], preferred_element_type=jnp.float32) + b1_ref[...], 0.0)
    z = jnp.dot(z, w2_ref[...], preferred_element_type=jnp.float32) + b2_ref[...]
    hb = jnp.maximum(z, 0.0)
    b = bt_ref[0, 0, :]
    onehot = (lax.broadcasted_iota(jnp.int32, (G, BT), 0) == b[None, :]).astype(jnp.float32)
    part = jnp.dot(onehot, hb, preferred_element_type=jnp.float32)

    @pl.when(i == 0)
    def _():
        o_ref[...] = part

    @pl.when(i > 0)
    def _():
        o_ref[...] += part


def _mlp_pool(h, agg, W1, b1, W2, b2, batch):
    return pl.pallas_call(
        _mlp_pool_body,
        grid=(NB,),
        in_specs=[
            pl.BlockSpec((BT, D), lambda i: (i, 0)),
            pl.BlockSpec((NC, BT, D), lambda i: (0, i, 0)),
            pl.BlockSpec((D, D), lambda i: (0, 0)),
            pl.BlockSpec((1, D), lambda i: (0, 0)),
            pl.BlockSpec((D, D), lambda i: (0, 0)),
            pl.BlockSpec((1, D), lambda i: (0, 0)),
            pl.BlockSpec((1, 1, BT), lambda i: (i, 0, 0)),
        ],
        out_specs=pl.BlockSpec((G, D), lambda i: (0, 0)),
        out_shape=jax.ShapeDtypeStruct((G, D), jnp.float32),
        compiler_params=pltpu.CompilerParams(dimension_semantics=("arbitrary",)),
    )(h, agg, W1, b1.reshape(1, D), W2, b2.reshape(1, D), batch.reshape(NB, 1, BT))


def kernel(x, edge_index, batch, W1_0, b1_0, W2_0, b2_0, W1_1, b1_1, W2_1, b2_1, W1_2, b1_2, W2_2, b2_2):
    src2 = edge_index[0].reshape(N_CHUNKS, CHUNK)
    dst2 = edge_index[1].reshape(N_CHUNKS, CHUNK)
    srcb = src2[: NW * BASE_CHUNKS].reshape(NW, NBLK, BPC, CHUNK)
    dstb = dst2[: NW * BASE_CHUNKS].reshape(NW, NBLK, BPC, CHUNK)
    ext = jnp.stack([src2[NW * BASE_CHUNKS:], dst2[NW * BASE_CHUNKS:]], axis=1)
    params = [(W1_0, b1_0, W2_0, b2_0), (W1_1, b1_1, W2_1, b2_1), (W1_2, b1_2, W2_2, b2_2)]
    h = x
    for li, (W1, b1, W2, b2) in enumerate(params):
        agg = _sc_agg(h, srcb, dstb, ext)
        if li < 2:
            h = _mlp(h, agg, W1, b1, W2, b2)
        else:
            out = _mlp_pool(h, agg, W1, b1, W2, b2, batch)
    return out


# speedup vs baseline: 9.3146x; 1.0438x over previous
import functools

import numpy as np

import jax
import jax.numpy as jnp
from jax import lax
from jax.experimental import pallas as pl
from jax.experimental.pallas import tpu as pltpu
from jax.experimental.pallas import tpu_sc as plsc

N = 10000
E = 320000
D = 128
G = 64

NC = 2
NS = 16
NW = NC * NS

CHUNK = 128
N_CHUNKS = E // CHUNK
BASE_CHUNKS = N_CHUNKS // NW
N_EXTRA = N_CHUNKS - BASE_CHUNKS * NW

ROWS_MAIN = (N // NS) // 8 * 8
TAIL_ROWS = N - NS * ROWS_MAIN

NBLK = 3
BPC = BASE_CHUNKS // NBLK
PAIRS = BPC // 2
BT = 2000
NB = N // BT
BTP = 1000
NBP = N // BTP


def _sc_agg_body(h_hbm, srcb_hbm, dstb_hbm, ext_hbm, out_hbm, s_u, s_v, d_u, d_v,
                 rows_a, rows_b, agg_sh, sem_i, sem_ga, sem_gb, sem_sa, sem_sb):
    c = lax.axis_index("c")
    s = lax.axis_index("s")
    w = c * NS + s

    def _gather(idx_row, buf, sem):
        pltpu.async_copy(h_hbm.at[idx_row], buf, sem)

    def _scatter(idx_row, buf, sem):
        pltpu.async_copy(buf, agg_sh.at[idx_row], sem, add=True)

    def _wait_g(idx_row, buf, sem):
        pltpu.make_async_copy(h_hbm.at[idx_row], buf, sem).wait()

    def _wait_s(idx_row, buf, sem):
        pltpu.make_async_copy(buf, agg_sh.at[idx_row], sem).wait()

    pltpu.sync_copy(srcb_hbm.at[w, 0], s_u)
    pltpu.sync_copy(dstb_hbm.at[w, 0], d_u)
    _gather(s_u.at[0], rows_a, sem_ga)

    def _zero_row(i, carry):
        for j in range(D // 16):
            rows_b[i, pl.ds(j * 16, 16)] = jnp.zeros((16,), jnp.float32)
        return carry

    lax.fori_loop(0, CHUNK, _zero_row, 0)
    done = 0
    while done < ROWS_MAIN:
        step = min(CHUNK, ROWS_MAIN - done)
        pltpu.sync_copy(rows_b.at[pl.ds(0, step)],
                        agg_sh.at[pl.ds(s * ROWS_MAIN + done, step)])
        done += step

    @pl.when(s == NS - 1)
    def _():
        pltpu.sync_copy(rows_b.at[pl.ds(0, TAIL_ROWS)], agg_sh.at[pl.ds(NS * ROWS_MAIN, TAIL_ROWS)])

    plsc.subcore_barrier()

    for b in range(NBLK):
        sb, db = ((s_u, d_u), (s_v, d_v))[b % 2]
        nsb, ndb = ((s_v, d_v), (s_u, d_u))[b % 2]

        def _pair(k, carry, sb=sb, db=db, nsb=nsb, ndb=ndb, b=b):
            def _drain_b():
                _wait_s(db.at[1], rows_b, sem_sb)

            if b == 0:
                pl.when(k > 0)(_drain_b)
            else:
                _drain_b()
            _gather(sb.at[2 * k + 1], rows_b, sem_gb)
            _wait_g(sb.at[2 * k], rows_a, sem_ga)
            _scatter(db.at[2 * k], rows_a, sem_sa)
            _wait_g(sb.at[2 * k + 1], rows_b, sem_gb)
            _scatter(db.at[2 * k + 1], rows_b, sem_sb)
            _wait_s(db.at[0], rows_a, sem_sa)

            @pl.when(k < PAIRS - 1)
            def _():
                _gather(sb.at[2 * k + 2], rows_a, sem_ga)

            if b < NBLK - 1:
                @pl.when(k == 0)
                def _():
                    pltpu.async_copy(srcb_hbm.at[w, b + 1], nsb, sem_i)
                    pltpu.async_copy(dstb_hbm.at[w, b + 1], ndb, sem_i)
            return carry

        lax.fori_loop(0, PAIRS, _pair, 0)
        if b < NBLK - 1:
            pltpu.make_async_copy(srcb_hbm.at[w, b + 1], nsb, sem_i).wait()
            pltpu.make_async_copy(dstb_hbm.at[w, b + 1], ndb, sem_i).wait()
            _gather(nsb.at[0], rows_a, sem_ga)

    _, last_db = ((s_u, d_u), (s_v, d_v))[(NBLK - 1) % 2]
    _wait_s(last_db.at[1], rows_b, sem_sb)

    @pl.when(w < N_EXTRA)
    def _():
        pltpu.sync_copy(ext_hbm.at[w], s_v.at[pl.ds(0, 2)])
        _gather(s_v.at[0], rows_a, sem_ga)
        _wait_g(s_v.at[0], rows_a, sem_ga)
        _scatter(s_v.at[1], rows_a, sem_sa)
        _wait_s(s_v.at[1], rows_a, sem_sa)

    plsc.subcore_barrier()

    pltpu.sync_copy(
        agg_sh.at[pl.ds(s * ROWS_MAIN, ROWS_MAIN)],
        out_hbm.at[c, pl.ds(s * ROWS_MAIN, ROWS_MAIN)],
    )

    @pl.when(s == NS - 1)
    def _():
        pltpu.sync_copy(
            agg_sh.at[pl.ds(NS * ROWS_MAIN, TAIL_ROWS)],
            out_hbm.at[c, pl.ds(NS * ROWS_MAIN, TAIL_ROWS)],
        )


def _sc_agg(h, srcb, dstb, ext):
    mesh = plsc.VectorSubcoreMesh(core_axis_name="c", subcore_axis_name="s")
    return pl.kernel(
        _sc_agg_body,
        out_type=jax.ShapeDtypeStruct((NC, N, D), jnp.float32),
        mesh=mesh,
        scratch_types=[
            pltpu.VMEM((BPC, CHUNK), jnp.int32),
            pltpu.VMEM((BPC, CHUNK), jnp.int32),
            pltpu.VMEM((BPC, CHUNK), jnp.int32),
            pltpu.VMEM((BPC, CHUNK), jnp.int32),
            pltpu.VMEM((CHUNK, D), jnp.float32),
            pltpu.VMEM((CHUNK, D), jnp.float32),
            pltpu.VMEM_SHARED((N, D), jnp.float32),
            pltpu.SemaphoreType.DMA,
            pltpu.SemaphoreType.DMA,
            pltpu.SemaphoreType.DMA,
            pltpu.SemaphoreType.DMA,
            pltpu.SemaphoreType.DMA,
        ],
    )(h, srcb, dstb, ext)


def _mlp_body(h_ref, a_ref, w1_ref, b1_ref, w2_ref, b2_ref, o_ref):
    z = h_ref[...] + a_ref[0] + a_ref[1]
    z = jnp.maximum(jnp.dot(z, w1_ref[...], preferred_element_type=jnp.float32) + b1_ref[...], 0.0)
    z = jnp.dot(z, w2_ref[...], preferred_element_type=jnp.float32) + b2_ref[...]
    o_ref[...] = jnp.maximum(z, 0.0)


def _mlp(h, agg, W1, b1, W2, b2):
    return pl.pallas_call(
        _mlp_body,
        grid=(NB,),
        in_specs=[
            pl.BlockSpec((BT, D), lambda i: (i, 0)),
            pl.BlockSpec((NC, BT, D), lambda i: (0, i, 0)),
            pl.BlockSpec((D, D), lambda i: (0, 0)),
            pl.BlockSpec((1, D), lambda i: (0, 0)),
            pl.BlockSpec((D, D), lambda i: (0, 0)),
            pl.BlockSpec((1, D), lambda i: (0, 0)),
        ],
        out_specs=pl.BlockSpec((BT, D), lambda i: (i, 0)),
        out_shape=jax.ShapeDtypeStruct((N, D), jnp.float32),
    )(h, agg, W1, b1.reshape(1, D), W2, b2.reshape(1, D))


def _mlp_pool_body(h_ref, a_ref, w1_ref, b1_ref, w2_ref, b2_ref, bt_ref, o_ref):
    i = pl.program_id(0)
    z = h_ref[...] + a_ref[0] + a_ref[1]
    z = jnp.maximum(jnp.dot(z, w1_ref[...], preferred_element_type=jnp.float32) + b1_ref[...], 0.0)
    z = jnp.dot(z, w2_ref[...], preferred_element_type=jnp.float32) + b2_ref[...]
    hb = jnp.maximum(z, 0.0)
    b = bt_ref[0, 0, :]
    onehot = (lax.broadcasted_iota(jnp.int32, (G, BTP), 0) == b[None, :]).astype(jnp.float32)
    part = jnp.dot(onehot, hb, preferred_element_type=jnp.float32)

    @pl.when(i == 0)
    def _():
        o_ref[...] = part

    @pl.when(i > 0)
    def _():
        o_ref[...] += part


def _mlp_pool(h, agg, W1, b1, W2, b2, batch):
    return pl.pallas_call(
        _mlp_pool_body,
        grid=(NBP,),
        in_specs=[
            pl.BlockSpec((BTP, D), lambda i: (i, 0)),
            pl.BlockSpec((NC, BTP, D), lambda i: (0, i, 0)),
            pl.BlockSpec((D, D), lambda i: (0, 0)),
            pl.BlockSpec((1, D), lambda i: (0, 0)),
            pl.BlockSpec((D, D), lambda i: (0, 0)),
            pl.BlockSpec((1, D), lambda i: (0, 0)),
            pl.BlockSpec((1, 1, BTP), lambda i: (i, 0, 0)),
        ],
        out_specs=pl.BlockSpec((G, D), lambda i: (0, 0)),
        out_shape=jax.ShapeDtypeStruct((G, D), jnp.float32),
        compiler_params=pltpu.CompilerParams(dimension_semantics=("arbitrary",)),
    )(h, agg, W1, b1.reshape(1, D), W2, b2.reshape(1, D), batch.reshape(NBP, 1, BTP))


def kernel(x, edge_index, batch, W1_0, b1_0, W2_0, b2_0, W1_1, b1_1, W2_1, b2_1, W1_2, b1_2, W2_2, b2_2):
    src2 = edge_index[0].reshape(N_CHUNKS, CHUNK)
    dst2 = edge_index[1].reshape(N_CHUNKS, CHUNK)
    srcb = src2[: NW * BASE_CHUNKS].reshape(NW, NBLK, BPC, CHUNK)
    dstb = dst2[: NW * BASE_CHUNKS].reshape(NW, NBLK, BPC, CHUNK)
    ext = jnp.stack([src2[NW * BASE_CHUNKS:], dst2[NW * BASE_CHUNKS:]], axis=1)
    params = [(W1_0, b1_0, W2_0, b2_0), (W1_1, b1_1, W2_1, b2_1), (W1_2, b1_2, W2_2, b2_2)]
    h = x
    for li, (W1, b1, W2, b2) in enumerate(params):
        agg = _sc_agg(h, srcb, dstb, ext)
        if li < 2:
            h = _mlp(h, agg, W1, b1, W2, b2)
        else:
            out = _mlp_pool(h, agg, W1, b1, W2, b2, batch)
    return out


# speedup vs baseline: 9.3990x; 1.0091x over previous
import functools

import numpy as np

import jax
import jax.numpy as jnp
from jax import lax
from jax.experimental import pallas as pl
from jax.experimental.pallas import tpu as pltpu
from jax.experimental.pallas import tpu_sc as plsc

N = 10000
E = 320000
D = 128
G = 64

NC = 2
NS = 16
NW = NC * NS

CHUNK = 128
N_CHUNKS = E // CHUNK
BASE_CHUNKS = N_CHUNKS // NW
N_EXTRA = N_CHUNKS - BASE_CHUNKS * NW

ROWS_MAIN = (N // NS) // 8 * 8
TAIL_ROWS = N - NS * ROWS_MAIN

NBLK = 3
BPC = BASE_CHUNKS // NBLK
PAIRS = BPC // 2
BT = 2000
NB = N // BT
BTP = 2000
NBP = N // BTP


def _sc_agg_body(h_hbm, srcb_hbm, dstb_hbm, ext_hbm, out_hbm, s_u, s_v, d_u, d_v,
                 rows_a, rows_b, agg_sh, sem_i, sem_ga, sem_gb, sem_sa, sem_sb):
    c = lax.axis_index("c")
    s = lax.axis_index("s")
    w = c * NS + s

    def _gather(idx_row, buf, sem):
        pltpu.async_copy(h_hbm.at[idx_row], buf, sem)

    def _scatter(idx_row, buf, sem):
        pltpu.async_copy(buf, agg_sh.at[idx_row], sem, add=True)

    def _wait_g(idx_row, buf, sem):
        pltpu.make_async_copy(h_hbm.at[idx_row], buf, sem).wait()

    def _wait_s(idx_row, buf, sem):
        pltpu.make_async_copy(buf, agg_sh.at[idx_row], sem).wait()

    pltpu.sync_copy(srcb_hbm.at[w, 0], s_u)
    pltpu.sync_copy(dstb_hbm.at[w, 0], d_u)
    _gather(s_u.at[0], rows_a, sem_ga)

    def _zero_row(i, carry):
        for j in range(D // 16):
            rows_b[i, pl.ds(j * 16, 16)] = jnp.zeros((16,), jnp.float32)
        return carry

    lax.fori_loop(0, CHUNK, _zero_row, 0)
    done = 0
    while done < ROWS_MAIN:
        step = min(CHUNK, ROWS_MAIN - done)
        pltpu.sync_copy(rows_b.at[pl.ds(0, step)],
                        agg_sh.at[pl.ds(s * ROWS_MAIN + done, step)])
        done += step

    @pl.when(s == NS - 1)
    def _():
        pltpu.sync_copy(rows_b.at[pl.ds(0, TAIL_ROWS)], agg_sh.at[pl.ds(NS * ROWS_MAIN, TAIL_ROWS)])

    plsc.subcore_barrier()

    for b in range(NBLK):
        sb, db = ((s_u, d_u), (s_v, d_v))[b % 2]
        nsb, ndb = ((s_v, d_v), (s_u, d_u))[b % 2]

        def _pair(k, carry, sb=sb, db=db, nsb=nsb, ndb=ndb, b=b):
            def _drain_b():
                _wait_s(db.at[1], rows_b, sem_sb)

            if b == 0:
                pl.when(k > 0)(_drain_b)
            else:
                _drain_b()
            _gather(sb.at[2 * k + 1], rows_b, sem_gb)
            _wait_g(sb.at[2 * k], rows_a, sem_ga)
            _scatter(db.at[2 * k], rows_a, sem_sa)
            _wait_g(sb.at[2 * k + 1], rows_b, sem_gb)
            _scatter(db.at[2 * k + 1], rows_b, sem_sb)
            _wait_s(db.at[0], rows_a, sem_sa)

            @pl.when(k < PAIRS - 1)
            def _():
                _gather(sb.at[2 * k + 2], rows_a, sem_ga)

            if b < NBLK - 1:
                @pl.when(k == 0)
                def _():
                    pltpu.async_copy(srcb_hbm.at[w, b + 1], nsb, sem_i)
                    pltpu.async_copy(dstb_hbm.at[w, b + 1], ndb, sem_i)
            return carry

        lax.fori_loop(0, PAIRS, _pair, 0)
        if b < NBLK - 1:
            pltpu.make_async_copy(srcb_hbm.at[w, b + 1], nsb, sem_i).wait()
            pltpu.make_async_copy(dstb_hbm.at[w, b + 1], ndb, sem_i).wait()
            _gather(nsb.at[0], rows_a, sem_ga)

    _, last_db = ((s_u, d_u), (s_v, d_v))[(NBLK - 1) % 2]
    _wait_s(last_db.at[1], rows_b, sem_sb)

    @pl.when(w < N_EXTRA)
    def _():
        pltpu.sync_copy(ext_hbm.at[w], s_v.at[pl.ds(0, 2)])
        _gather(s_v.at[0], rows_a, sem_ga)
        _wait_g(s_v.at[0], rows_a, sem_ga)
        _scatter(s_v.at[1], rows_a, sem_sa)
        _wait_s(s_v.at[1], rows_a, sem_sa)

    plsc.subcore_barrier()

    pltpu.sync_copy(
        agg_sh.at[pl.ds(s * ROWS_MAIN, ROWS_MAIN)],
        out_hbm.at[c, pl.ds(s * ROWS_MAIN, ROWS_MAIN)],
    )

    @pl.when(s == NS - 1)
    def _():
        pltpu.sync_copy(
            agg_sh.at[pl.ds(NS * ROWS_MAIN, TAIL_ROWS)],
            out_hbm.at[c, pl.ds(NS * ROWS_MAIN, TAIL_ROWS)],
        )


def _sc_agg(h, srcb, dstb, ext):
    mesh = plsc.VectorSubcoreMesh(core_axis_name="c", subcore_axis_name="s")
    return pl.kernel(
        _sc_agg_body,
        out_type=jax.ShapeDtypeStruct((NC, N, D), jnp.float32),
        mesh=mesh,
        scratch_types=[
            pltpu.VMEM((BPC, CHUNK), jnp.int32),
            pltpu.VMEM((BPC, CHUNK), jnp.int32),
            pltpu.VMEM((BPC, CHUNK), jnp.int32),
            pltpu.VMEM((BPC, CHUNK), jnp.int32),
            pltpu.VMEM((CHUNK, D), jnp.float32),
            pltpu.VMEM((CHUNK, D), jnp.float32),
            pltpu.VMEM_SHARED((N, D), jnp.float32),
            pltpu.SemaphoreType.DMA,
            pltpu.SemaphoreType.DMA,
            pltpu.SemaphoreType.DMA,
            pltpu.SemaphoreType.DMA,
            pltpu.SemaphoreType.DMA,
        ],
    )(h, srcb, dstb, ext)


def _mlp_body(h_ref, a_ref, w1_ref, b1_ref, w2_ref, b2_ref, o_ref):
    z = h_ref[...] + a_ref[0] + a_ref[1]
    z = jnp.maximum(jnp.dot(z, w1_ref[...], preferred_element_type=jnp.float32) + b1_ref[...], 0.0)
    z = jnp.dot(z, w2_ref[...], preferred_element_type=jnp.float32) + b2_ref[...]
    o_ref[...] = jnp.maximum(z, 0.0)


def _mlp(h, agg, W1, b1, W2, b2):
    return pl.pallas_call(
        _mlp_body,
        grid=(NB,),
        in_specs=[
            pl.BlockSpec((BT, D), lambda i: (i, 0)),
            pl.BlockSpec((NC, BT, D), lambda i: (0, i, 0)),
            pl.BlockSpec((D, D), lambda i: (0, 0)),
            pl.BlockSpec((1, D), lambda i: (0, 0)),
            pl.BlockSpec((D, D), lambda i: (0, 0)),
            pl.BlockSpec((1, D), lambda i: (0, 0)),
        ],
        out_specs=pl.BlockSpec((BT, D), lambda i: (i, 0)),
        out_shape=jax.ShapeDtypeStruct((N, D), jnp.float32),
    )(h, agg, W1, b1.reshape(1, D), W2, b2.reshape(1, D))


def _mlp_pool_body(h_ref, a_ref, w1_ref, b1_ref, w2_ref, b2_ref, bt_ref, o_ref):
    i = pl.program_id(0)
    z = h_ref[...] + a_ref[0] + a_ref[1]
    z = jnp.maximum(jnp.dot(z, w1_ref[...], preferred_element_type=jnp.float32) + b1_ref[...], 0.0)
    z = jnp.dot(z, w2_ref[...], preferred_element_type=jnp.float32) + b2_ref[...]
    hb = jnp.maximum(z, 0.0)
    b = bt_ref[0, 0, :]
    onehot = (lax.broadcasted_iota(jnp.int32, (G, BTP), 0) == b[None, :]).astype(jnp.float32)
    part = jnp.dot(onehot, hb, preferred_element_type=jnp.float32)

    @pl.when(i == 0)
    def _():
        o_ref[...] = part

    @pl.when(i > 0)
    def _():
        o_ref[...] += part


def _mlp_pool(h, agg, W1, b1, W2, b2, batch):
    return pl.pallas_call(
        _mlp_pool_body,
        grid=(NBP,),
        in_specs=[
            pl.BlockSpec((BTP, D), lambda i: (i, 0)),
            pl.BlockSpec((NC, BTP, D), lambda i: (0, i, 0)),
            pl.BlockSpec((D, D), lambda i: (0, 0)),
            pl.BlockSpec((1, D), lambda i: (0, 0)),
            pl.BlockSpec((D, D), lambda i: (0, 0)),
            pl.BlockSpec((1, D), lambda i: (0, 0)),
            pl.BlockSpec((1, 1, BTP), lambda i: (i, 0, 0)),
        ],
        out_specs=pl.BlockSpec((G, D), lambda i: (0, 0)),
        out_shape=jax.ShapeDtypeStruct((G, D), jnp.float32),
        compiler_params=pltpu.CompilerParams(dimension_semantics=("arbitrary",)),
    )(h, agg, W1, b1.reshape(1, D), W2, b2.reshape(1, D), batch.reshape(NBP, 1, BTP))


def kernel(x, edge_index, batch, W1_0, b1_0, W2_0, b2_0, W1_1, b1_1, W2_1, b2_1, W1_2, b1_2, W2_2, b2_2):
    src2 = edge_index[0].reshape(N_CHUNKS, CHUNK)
    dst2 = edge_index[1].reshape(N_CHUNKS, CHUNK)
    srcb = src2[: NW * BASE_CHUNKS].reshape(NW, NBLK, BPC, CHUNK)
    dstb = dst2[: NW * BASE_CHUNKS].reshape(NW, NBLK, BPC, CHUNK)
    ext = jnp.stack([src2[NW * BASE_CHUNKS:], dst2[NW * BASE_CHUNKS:]], axis=1)
    params = [(W1_0, b1_0, W2_0, b2_0), (W1_1, b1_1, W2_1, b2_1), (W1_2, b1_2, W2_2, b2_2)]
    h = x
    for li, (W1, b1, W2, b2) in enumerate(params):
        agg = _sc_agg(h, srcb, dstb, ext)
        if li < 2:
            h = _mlp(h, agg, W1, b1, W2, b2)
        else:
            out = _mlp_pool(h, agg, W1, b1, W2, b2, batch)
    return out


# speedup vs baseline: 9.4133x; 1.0015x over previous
import jax
import jax.numpy as jnp
from jax import lax
from jax.experimental import pallas as pl
from jax.experimental.pallas import tpu as pltpu
from jax.experimental.pallas import tpu_sc as plsc

N = 10000
E = 320000
D = 128
G = 64

NC = 2
NS = 16
NW = NC * NS

CHUNK = 128
N_CHUNKS = E // CHUNK
BASE_CHUNKS = N_CHUNKS // NW
N_EXTRA = N_CHUNKS - BASE_CHUNKS * NW

ROWS_MAIN = (N // NS) // 8 * 8
TAIL_ROWS = N - NS * ROWS_MAIN

NBLK = 3
BPC = BASE_CHUNKS // NBLK
PAIRS = BPC // 2
BT = 2000
NB = N // BT
BTP = 2000
NBP = N // BTP


def _sc_agg_body(h_hbm, srcb_hbm, dstb_hbm, ext_hbm, out_hbm, s_u, s_v, d_u, d_v,
                 rows_a, rows_b, agg_sh, sem_i, sem_ga, sem_gb, sem_sa, sem_sb):
    c = lax.axis_index("c")
    s = lax.axis_index("s")
    w = c * NS + s

    def _gather(idx_row, buf, sem):
        pltpu.async_copy(h_hbm.at[idx_row], buf, sem)

    def _scatter(idx_row, buf, sem):
        pltpu.async_copy(buf, agg_sh.at[idx_row], sem, add=True)

    def _wait_g(idx_row, buf, sem):
        pltpu.make_async_copy(h_hbm.at[idx_row], buf, sem).wait()

    def _wait_s(idx_row, buf, sem):
        pltpu.make_async_copy(buf, agg_sh.at[idx_row], sem).wait()

    pltpu.sync_copy(srcb_hbm.at[w, 0], s_u)
    pltpu.sync_copy(dstb_hbm.at[w, 0], d_u)
    _gather(s_u.at[0], rows_a, sem_ga)

    def _zero_row(i, carry):
        for j in range(D // 16):
            rows_b[i, pl.ds(j * 16, 16)] = jnp.zeros((16,), jnp.float32)
        return carry

    lax.fori_loop(0, CHUNK, _zero_row, 0)
    done = 0
    while done < ROWS_MAIN:
        step = min(CHUNK, ROWS_MAIN - done)
        pltpu.sync_copy(rows_b.at[pl.ds(0, step)],
                        agg_sh.at[pl.ds(s * ROWS_MAIN + done, step)])
        done += step

    @pl.when(s == NS - 1)
    def _():
        pltpu.sync_copy(rows_b.at[pl.ds(0, TAIL_ROWS)], agg_sh.at[pl.ds(NS * ROWS_MAIN, TAIL_ROWS)])

    plsc.subcore_barrier()

    for b in range(NBLK):
        sb, db = ((s_u, d_u), (s_v, d_v))[b % 2]
        nsb, ndb = ((s_v, d_v), (s_u, d_u))[b % 2]

        def _pair(k, carry, sb=sb, db=db, nsb=nsb, ndb=ndb, b=b):
            def _drain_b():
                _wait_s(db.at[1], rows_b, sem_sb)

            if b == 0:
                pl.when(k > 0)(_drain_b)
            else:
                _drain_b()
            _gather(sb.at[2 * k + 1], rows_b, sem_gb)
            _wait_g(sb.at[2 * k], rows_a, sem_ga)
            _scatter(db.at[2 * k], rows_a, sem_sa)
            _wait_g(sb.at[2 * k + 1], rows_b, sem_gb)
            _scatter(db.at[2 * k + 1], rows_b, sem_sb)
            _wait_s(db.at[0], rows_a, sem_sa)

            @pl.when(k < PAIRS - 1)
            def _():
                _gather(sb.at[2 * k + 2], rows_a, sem_ga)

            if b < NBLK - 1:
                @pl.when(k == 0)
                def _():
                    pltpu.async_copy(srcb_hbm.at[w, b + 1], nsb, sem_i)
                    pltpu.async_copy(dstb_hbm.at[w, b + 1], ndb, sem_i)
            return carry

        lax.fori_loop(0, PAIRS, _pair, 0)
        if b < NBLK - 1:
            pltpu.make_async_copy(srcb_hbm.at[w, b + 1], nsb, sem_i).wait()
            pltpu.make_async_copy(dstb_hbm.at[w, b + 1], ndb, sem_i).wait()
            _gather(nsb.at[0], rows_a, sem_ga)

    _, last_db = ((s_u, d_u), (s_v, d_v))[(NBLK - 1) % 2]
    _wait_s(last_db.at[1], rows_b, sem_sb)

    @pl.when(w < N_EXTRA)
    def _():
        pltpu.sync_copy(ext_hbm.at[w], s_v.at[pl.ds(0, 2)])
        _gather(s_v.at[0], rows_a, sem_ga)
        _wait_g(s_v.at[0], rows_a, sem_ga)
        _scatter(s_v.at[1], rows_a, sem_sa)
        _wait_s(s_v.at[1], rows_a, sem_sa)

    plsc.subcore_barrier()

    pltpu.sync_copy(
        agg_sh.at[pl.ds(s * ROWS_MAIN, ROWS_MAIN)],
        out_hbm.at[c, pl.ds(s * ROWS_MAIN, ROWS_MAIN)],
    )

    @pl.when(s == NS - 1)
    def _():
        pltpu.sync_copy(
            agg_sh.at[pl.ds(NS * ROWS_MAIN, TAIL_ROWS)],
            out_hbm.at[c, pl.ds(NS * ROWS_MAIN, TAIL_ROWS)],
        )


def _sc_agg(h, srcb, dstb, ext):
    mesh = plsc.VectorSubcoreMesh(core_axis_name="c", subcore_axis_name="s")
    return pl.kernel(
        _sc_agg_body,
        out_type=jax.ShapeDtypeStruct((NC, N, D), jnp.float32),
        mesh=mesh,
        scratch_types=[
            pltpu.VMEM((BPC, CHUNK), jnp.int32),
            pltpu.VMEM((BPC, CHUNK), jnp.int32),
            pltpu.VMEM((BPC, CHUNK), jnp.int32),
            pltpu.VMEM((BPC, CHUNK), jnp.int32),
            pltpu.VMEM((CHUNK, D), jnp.float32),
            pltpu.VMEM((CHUNK, D), jnp.float32),
            pltpu.VMEM_SHARED((N, D), jnp.float32),
            pltpu.SemaphoreType.DMA,
            pltpu.SemaphoreType.DMA,
            pltpu.SemaphoreType.DMA,
            pltpu.SemaphoreType.DMA,
            pltpu.SemaphoreType.DMA,
        ],
    )(h, srcb, dstb, ext)


def _mlp_body(h_ref, a_ref, w1_ref, b1_ref, w2_ref, b2_ref, o_ref):
    z = h_ref[...] + a_ref[0] + a_ref[1]
    z = jnp.maximum(jnp.dot(z, w1_ref[...], preferred_element_type=jnp.float32) + b1_ref[...], 0.0)
    z = jnp.dot(z, w2_ref[...], preferred_element_type=jnp.float32) + b2_ref[...]
    o_ref[...] = jnp.maximum(z, 0.0)


def _mlp(h, agg, W1, b1, W2, b2):
    return pl.pallas_call(
        _mlp_body,
        grid=(NB,),
        in_specs=[
            pl.BlockSpec((BT, D), lambda i: (i, 0)),
            pl.BlockSpec((NC, BT, D), lambda i: (0, i, 0)),
            pl.BlockSpec((D, D), lambda i: (0, 0)),
            pl.BlockSpec((1, D), lambda i: (0, 0)),
            pl.BlockSpec((D, D), lambda i: (0, 0)),
            pl.BlockSpec((1, D), lambda i: (0, 0)),
        ],
        out_specs=pl.BlockSpec((BT, D), lambda i: (i, 0)),
        out_shape=jax.ShapeDtypeStruct((N, D), jnp.float32),
    )(h, agg, W1, b1.reshape(1, D), W2, b2.reshape(1, D))


def _mlp_pool_body(h_ref, a_ref, w1_ref, b1_ref, w2_ref, b2_ref, bt_ref, o_ref):
    i = pl.program_id(0)
    z = h_ref[...] + a_ref[0] + a_ref[1]
    z = jnp.maximum(jnp.dot(z, w1_ref[...], preferred_element_type=jnp.float32) + b1_ref[...], 0.0)
    z = jnp.dot(z, w2_ref[...], preferred_element_type=jnp.float32) + b2_ref[...]
    hb = jnp.maximum(z, 0.0)
    b = bt_ref[0, 0, :]
    onehot = (lax.broadcasted_iota(jnp.int32, (G, BTP), 0) == b[None, :]).astype(jnp.float32)
    part = jnp.dot(onehot, hb, preferred_element_type=jnp.float32)

    @pl.when(i == 0)
    def _():
        o_ref[...] = part

    @pl.when(i > 0)
    def _():
        o_ref[...] += part


def _mlp_pool(h, agg, W1, b1, W2, b2, batch):
    return pl.pallas_call(
        _mlp_pool_body,
        grid=(NBP,),
        in_specs=[
            pl.BlockSpec((BTP, D), lambda i: (i, 0)),
            pl.BlockSpec((NC, BTP, D), lambda i: (0, i, 0)),
            pl.BlockSpec((D, D), lambda i: (0, 0)),
            pl.BlockSpec((1, D), lambda i: (0, 0)),
            pl.BlockSpec((D, D), lambda i: (0, 0)),
            pl.BlockSpec((1, D), lambda i: (0, 0)),
            pl.BlockSpec((1, 1, BTP), lambda i: (i, 0, 0)),
        ],
        out_specs=pl.BlockSpec((G, D), lambda i: (0, 0)),
        out_shape=jax.ShapeDtypeStruct((G, D), jnp.float32),
        compiler_params=pltpu.CompilerParams(dimension_semantics=("arbitrary",)),
    )(h, agg, W1, b1.reshape(1, D), W2, b2.reshape(1, D), batch.reshape(NBP, 1, BTP))


def kernel(x, edge_index, batch, W1_0, b1_0, W2_0, b2_0, W1_1, b1_1, W2_1, b2_1, W1_2, b1_2, W2_2, b2_2):
    src2 = edge_index[0].reshape(N_CHUNKS, CHUNK)
    dst2 = edge_index[1].reshape(N_CHUNKS, CHUNK)
    srcb = src2[: NW * BASE_CHUNKS].reshape(NW, NBLK, BPC, CHUNK)
    dstb = dst2[: NW * BASE_CHUNKS].reshape(NW, NBLK, BPC, CHUNK)
    ext = jnp.stack([src2[NW * BASE_CHUNKS:], dst2[NW * BASE_CHUNKS:]], axis=1)
    params = [(W1_0, b1_0, W2_0, b2_0), (W1_1, b1_1, W2_1, b2_1), (W1_2, b1_2, W2_2, b2_2)]
    h = x
    for li, (W1, b1, W2, b2) in enumerate(params):
        agg = _sc_agg(h, srcb, dstb, ext)
        if li < 2:
            h = _mlp(h, agg, W1, b1, W2, b2)
        else:
            out = _mlp_pool(h, agg, W1, b1, W2, b2, batch)
    return out
